# baseline (speedup 1.0000x reference)
import functools

import jax
import jax.numpy as jnp
from jax import lax
from jax.experimental import pallas as pl
from jax.experimental.pallas import tpu as pltpu
from jax.experimental.pallas import tpu_sc as plsc

DIM = 128
VPAD = 128
CHUNK = 256
IDX_ROWS = CHUNK // 128
NBUF = 3


def _proj_body(t_ref, w_ref, b_ref, o_ref):
    o_ref[...] = lax.dot_general(
        t_ref[...], w_ref[...],
        (((1,), (1,)), ((), ())),
        preferred_element_type=jnp.float32,
    ) + b_ref[...]


def _compute_proj(table_pad, W, b2):
    return pl.pallas_call(
        _proj_body,
        out_shape=jax.ShapeDtypeStruct((VPAD, DIM), jnp.float32),
    )(table_pad, W, b2)


@functools.lru_cache(maxsize=None)
def _make_gather(B):
    NW = 32
    bpw = B // NW
    n_chunks = bpw // CHUNK
    idx_rows_pw = bpw // 128
    mesh = plsc.VectorSubcoreMesh(core_axis_name="c", subcore_axis_name="s")

    @functools.partial(
        pl.kernel, mesh=mesh,
        out_type=jax.ShapeDtypeStruct((B, DIM), jnp.float32),
        scratch_types=[
            pltpu.VMEM_SHARED((VPAD, DIM), jnp.float32),
            pltpu.VMEM((NBUF, IDX_ROWS, 128), jnp.int32),
            pltpu.VMEM((NBUF, CHUNK, DIM), jnp.float32),
        ]
        + [pltpu.SemaphoreType.DMA] * (IDX_ROWS + 2 * NBUF),
    )
    def k(proj_hbm, idx_hbm, out_hbm, table_v, idx_v, rows_v, *sems):
        gsems = sems[:IDX_ROWS]
        wsems = sems[IDX_ROWS:IDX_ROWS + NBUF]
        isems = sems[IDX_ROWS + NBUF:]
        wid = lax.axis_index("s") * 2 + lax.axis_index("c")
        row0 = wid * idx_rows_pw
        base0 = wid * bpw
        max_row0 = (B // 128) - IDX_ROWS

        @pl.when(lax.axis_index("s") == 0)
        def _():
            pltpu.sync_copy(proj_hbm, table_v)

        plsc.subcore_barrier()

        def start_idx(i, s):
            row = jnp.minimum(row0 + i * IDX_ROWS, max_row0)
            pltpu.async_copy(
                idx_hbm.at[pl.ds(row, IDX_ROWS)], idx_v.at[s], isems[s])

        def wait_idx(s):
            pltpu.make_async_copy(
                idx_hbm.at[pl.ds(0, IDX_ROWS)], idx_v.at[s], isems[s]).wait()

        def gather_and_write(i, s):
            copies = [
                pltpu.async_copy(
                    table_v.at[idx_v.at[s, j]],
                    rows_v.at[s, pl.ds(j * 128, 128)], gsems[j])
                for j in range(IDX_ROWS)
            ]
            for j, c in enumerate(copies):
                c.wait()
                pltpu.async_copy(
                    rows_v.at[s, pl.ds(j * 128, 128)],
                    out_hbm.at[pl.ds(base0 + i * CHUNK + j * 128, 128)],
                    wsems[s])

        def wait_write(s):
            pltpu.make_async_copy(
                rows_v.at[s], out_hbm.at[pl.ds(base0, CHUNK)],
                wsems[s]).wait()

        for s in range(NBUF):
            start_idx(s, s)
        for s in range(NBUF):
            wait_idx(s)
            gather_and_write(s, s)
            start_idx(s + NBUF, s)

        def body(t, carry):
            for s in range(NBUF):
                i = t * NBUF + s
                wait_write(s)
                wait_idx(s)
                gather_and_write(i, s)
                start_idx(i + NBUF, s)
            return carry

        lax.fori_loop(1, n_chunks // NBUF, body, 0)
        for i in range((n_chunks // NBUF) * NBUF, n_chunks):
            s = i % NBUF
            wait_write(s)
            wait_idx(s)
            gather_and_write(i, s)
            start_idx(i + NBUF, s)
        for s in range(NBUF):
            wait_write(s)
            wait_idx(s)

    return k


def kernel(x, emb_table, W, b):
    Bsz, L = x.shape
    V, D = emb_table.shape
    table_pad = jnp.zeros((VPAD, D), emb_table.dtype).at[:V].set(emb_table)
    proj = _compute_proj(table_pad, W, b.reshape(1, D))
    xf = x.reshape(-1).astype(jnp.int32)
    B = xf.shape[0]
    idx2 = xf.reshape(B // 128, 128)
    out = _make_gather(B)(proj, idx2)
    return out.reshape(Bsz, L, D)


# speedup vs baseline: 1.0330x; 1.0330x over previous
import functools

import jax
import jax.numpy as jnp
from jax import lax
from jax.experimental import pallas as pl
from jax.experimental.pallas import tpu as pltpu
from jax.experimental.pallas import tpu_sc as plsc

DIM = 128
VPAD = 128
CHUNK = 256
IDX_ROWS = CHUNK // 128
NBUF = 3


def _proj_body(t_ref, w_ref, b_ref, o_ref):
    o_ref[...] = lax.dot_general(
        t_ref[...], w_ref[...],
        (((1,), (1,)), ((), ())),
        preferred_element_type=jnp.float32,
    ) + b_ref[...]


def _compute_proj(table_pad, W, b2):
    return pl.pallas_call(
        _proj_body,
        out_shape=jax.ShapeDtypeStruct((VPAD, DIM), jnp.float32),
    )(table_pad, W, b2)


@functools.lru_cache(maxsize=None)
def _make_gather(B):
    NW = 32
    bpw = B // NW
    n_chunks = bpw // CHUNK
    idx_rows_pw = bpw // 128
    mesh = plsc.VectorSubcoreMesh(core_axis_name="c", subcore_axis_name="s")

    @functools.partial(
        pl.kernel, mesh=mesh,
        out_type=jax.ShapeDtypeStruct((B, DIM), jnp.float32),
        scratch_types=[
            pltpu.VMEM_SHARED((VPAD, DIM), jnp.float32),
            pltpu.VMEM((NBUF, IDX_ROWS, 128), jnp.int32),
            pltpu.VMEM((NBUF, CHUNK, DIM), jnp.float32),
        ]
        + [pltpu.SemaphoreType.DMA] * (NBUF * IDX_ROWS + 2 * NBUF),
    )
    def k(proj_hbm, idx_hbm, out_hbm, table_v, idx_v, rows_v, *sems):
        gsems = [sems[s * IDX_ROWS:(s + 1) * IDX_ROWS] for s in range(NBUF)]
        wsems = sems[NBUF * IDX_ROWS:NBUF * IDX_ROWS + NBUF]
        isems = sems[NBUF * IDX_ROWS + NBUF:]
        wid = lax.axis_index("s") * 2 + lax.axis_index("c")
        row0 = wid * idx_rows_pw
        base0 = wid * bpw
        max_row0 = (B // 128) - IDX_ROWS

        @pl.when(lax.axis_index("s") == 0)
        def _():
            pltpu.sync_copy(proj_hbm, table_v)

        plsc.subcore_barrier()

        def start_idx(i, s):
            row = jnp.minimum(row0 + i * IDX_ROWS, max_row0)
            pltpu.async_copy(
                idx_hbm.at[pl.ds(row, IDX_ROWS)], idx_v.at[s], isems[s])

        def wait_idx(s):
            pltpu.make_async_copy(
                idx_hbm.at[pl.ds(0, IDX_ROWS)], idx_v.at[s], isems[s]).wait()

        def start_gathers(i, s):
            for j in range(IDX_ROWS):
                pltpu.async_copy(
                    table_v.at[idx_v.at[s, j]],
                    rows_v.at[s, pl.ds(j * 128, 128)], gsems[s][j])

        def finish_chunk(p, sp):
            for j in range(IDX_ROWS):
                pltpu.make_async_copy(
                    table_v.at[idx_v.at[sp, j]],
                    rows_v.at[sp, pl.ds(j * 128, 128)], gsems[sp][j]).wait()
                pltpu.async_copy(
                    rows_v.at[sp, pl.ds(j * 128, 128)],
                    out_hbm.at[pl.ds(base0 + p * CHUNK + j * 128, 128)],
                    wsems[sp])
            start_idx(p + NBUF, sp)

        def wait_write(s):
            pltpu.make_async_copy(
                rows_v.at[s], out_hbm.at[pl.ds(base0, CHUNK)],
                wsems[s]).wait()

        for s in range(NBUF):
            start_idx(s, s)
        for s in range(NBUF):
            wait_idx(s)
            start_gathers(s, s)
            if s >= 1:
                finish_chunk(s - 1, s - 1)

        def body(t, carry):
            for s in range(NBUF):
                i = t * NBUF + s
                wait_write(s)
                wait_idx(s)
                start_gathers(i, s)
                finish_chunk(i - 1, (s - 1) % NBUF)
            return carry

        lax.fori_loop(1, n_chunks // NBUF, body, 0)
        for i in range((n_chunks // NBUF) * NBUF, n_chunks):
            s = i % NBUF
            wait_write(s)
            wait_idx(s)
            start_gathers(i, s)
            finish_chunk(i - 1, (s - 1) % NBUF)
        finish_chunk(n_chunks - 1, (n_chunks - 1) % NBUF)
        for s in range(NBUF):
            wait_write(s)
            wait_idx(s)

    return k


def kernel(x, emb_table, W, b):
    Bsz, L = x.shape
    V, D = emb_table.shape
    table_pad = jnp.zeros((VPAD, D), emb_table.dtype).at[:V].set(emb_table)
    proj = _compute_proj(table_pad, W, b.reshape(1, D))
    xf = x.reshape(-1).astype(jnp.int32)
    B = xf.shape[0]
    idx2 = xf.reshape(B // 128, 128)
    out = _make_gather(B)(proj, idx2)
    return out.reshape(Bsz, L, D)
